# baseline (speedup 1.0000x reference)
import functools

import jax
import jax.numpy as jnp
from jax import lax
from jax.experimental import pallas as pl
from jax.experimental.pallas import tpu as pltpu
from jax.experimental.pallas import tpu_sc as plsc

N_ATOMS = 100000
EMB = 128
TABLE_ROWS = 101
CHUNK = 128
GSIZE = 16
PADEMB = EMB

_info = plsc.get_sparse_core_info()
NC = _info.num_cores
NS = _info.num_subcores
NW = NC * NS

CHUNKS_PER_W = -(-N_ATOMS // (CHUNK * NW))
SPAN = CHUNKS_PER_W * CHUNK
S_STREAM = 16
V_CHUNKS = CHUNKS_PER_W - S_STREAM


def _make_lookup():
    mesh = plsc.VectorSubcoreMesh(core_axis_name="c", subcore_axis_name="s")

    @functools.partial(
        pl.kernel,
        mesh=mesh,
        compiler_params=pltpu.CompilerParams(needs_layout_passes=False),
        out_type=jax.ShapeDtypeStruct((N_ATOMS, EMB), jnp.float32),
        scratch_types=[
            pltpu.VMEM((SPAN,), jnp.int32),
            pltpu.VMEM((CHUNK, EMB), jnp.float32),
            pltpu.VMEM((CHUNK, EMB), jnp.float32),
            pltpu.VMEM((CHUNK, PADEMB), jnp.float32),
            pltpu.VMEM((CHUNK, PADEMB), jnp.float32),
            pltpu.VMEM((TABLE_ROWS, PADEMB), jnp.float32),
            pltpu.VMEM_SHARED((TABLE_ROWS, EMB), jnp.float32),
            pltpu.SemaphoreType.DMA,
            pltpu.SemaphoreType.DMA,
            pltpu.SemaphoreType.DMA,
            pltpu.SemaphoreType.DMA,
            pltpu.SemaphoreType.DMA,
            pltpu.SemaphoreType.DMA,
        ],
    )
    def lookup(z_hbm, table_hbm, out_hbm, idx_all, rows0, rows1, vbuf0,
               vbuf1, table_loc, table_sh, gsem0, gsem1, wsem0, wsem1,
               vsem0, vsem1):
        sid = lax.axis_index("s")
        wid = sid * NC + lax.axis_index("c")

        @pl.when(sid == 0)
        def _():
            pltpu.sync_copy(table_hbm, table_sh.at[pl.ds(1, TABLE_ROWS - 1)])

        start = jnp.minimum(wid * SPAN, N_ATOMS - SPAN)
        pltpu.sync_copy(z_hbm.at[pl.ds(start, SPAN)], idx_all)

        plsc.subcore_barrier()

        rows = (rows0, rows1)
        vbuf = (vbuf0, vbuf1)
        gsem = (gsem0, gsem1)
        wsem = (wsem0, wsem1)
        vsem = (vsem0, vsem1)

        def issue_gather(k, b):
            pltpu.async_copy(
                table_sh.at[idx_all.at[pl.ds(k * CHUNK, CHUNK)]],
                rows[b], gsem[b])

        def drain(sem, buf):
            pltpu.make_async_copy(out_hbm.at[pl.ds(0, CHUNK)], buf,
                                  sem).wait()

        def stream_event(e):
            if e >= S_STREAM:
                return
            if e + 1 < S_STREAM:
                if e >= 1:
                    drain(wsem[(e - 1) % 2], rows[(e - 1) % 2])
                issue_gather(e + 1, (e + 1) % 2)
            drain(gsem[e % 2], rows[e % 2])
            pltpu.async_copy(rows[e % 2],
                             out_hbm.at[pl.ds(start + e * CHUNK, CHUNK)],
                             wsem[e % 2])

        def expand_group(kv, g, dst):
            z16 = idx_all[pl.ds(kv * CHUNK + g * GSIZE, GSIZE)]
            rowids = g * GSIZE + lax.iota(jnp.int32, GSIZE)

            @plsc.parallel_loop(0, EMB, 1, unroll=16)
            def _(c):
                cvec = jnp.zeros((GSIZE,), jnp.int32) + c
                vals = plsc.load_gather(table_loc, [z16, cvec])
                plsc.store_scatter(dst, [rowids, cvec], vals)

        def expand_half(kv, h, dst):
            ng = CHUNK // GSIZE

            def body(g, carry):
                expand_group(kv, g, dst)
                return carry

            lax.fori_loop(h * ng // 2, (h + 1) * ng // 2, body, 0)

        issue_gather(0, 0)
        pltpu.sync_copy(table_sh, table_loc.at[:, pl.ds(0, EMB)])

        for i in range(V_CHUNKS):
            kv = S_STREAM + i
            p = i % 2
            if i >= 2:
                drain(vsem[p], vbuf[p].at[:, pl.ds(0, EMB)])
            stream_event(2 * i)
            expand_half(kv, 0, vbuf[p])
            stream_event(2 * i + 1)
            expand_half(kv, 1, vbuf[p])
            pltpu.async_copy(vbuf[p].at[:, pl.ds(0, EMB)],
                             out_hbm.at[pl.ds(start + kv * CHUNK, CHUNK)],
                             vsem[p])

        for e in range(2 * V_CHUNKS, S_STREAM):
            stream_event(e)

        drain(wsem[(S_STREAM - 2) % 2], rows[(S_STREAM - 2) % 2])
        drain(wsem[(S_STREAM - 1) % 2], rows[(S_STREAM - 1) % 2])
        if V_CHUNKS >= 2:
            drain(vsem[(V_CHUNKS - 2) % 2],
                  vbuf[(V_CHUNKS - 2) % 2].at[:, pl.ds(0, EMB)])
        if V_CHUNKS >= 1:
            drain(vsem[(V_CHUNKS - 1) % 2],
                  vbuf[(V_CHUNKS - 1) % 2].at[:, pl.ds(0, EMB)])

    return lookup


_lookup = _make_lookup()


def kernel(Z, W):
    return _lookup(Z, W)


# speedup vs baseline: 1.5186x; 1.5186x over previous
import functools

import jax
import jax.numpy as jnp
from jax import lax
from jax.experimental import pallas as pl
from jax.experimental.pallas import tpu as pltpu
from jax.experimental.pallas import tpu_sc as plsc

N_ATOMS = 100000
EMB = 128
TABLE_ROWS = 101
CHUNK = 128
GSIZE = 16
PADEMB = EMB

_info = plsc.get_sparse_core_info()
NC = _info.num_cores
NS = _info.num_subcores
NW = NC * NS

CHUNKS_PER_W = -(-N_ATOMS // (CHUNK * NW))
SPAN = CHUNKS_PER_W * CHUNK
S_STREAM = 21
V_CHUNKS = CHUNKS_PER_W - S_STREAM


def _events_at_slot(j):
    nslots = 2 * V_CHUNKS
    return range((S_STREAM * j) // nslots, (S_STREAM * (j + 1)) // nslots)


def _make_lookup():
    mesh = plsc.VectorSubcoreMesh(core_axis_name="c", subcore_axis_name="s")

    @functools.partial(
        pl.kernel,
        mesh=mesh,
        compiler_params=pltpu.CompilerParams(needs_layout_passes=False),
        out_type=jax.ShapeDtypeStruct((N_ATOMS, EMB), jnp.float32),
        scratch_types=[
            pltpu.VMEM((SPAN,), jnp.int32),
            pltpu.VMEM((CHUNK, EMB), jnp.float32),
            pltpu.VMEM((CHUNK, EMB), jnp.float32),
            pltpu.VMEM((CHUNK, PADEMB), jnp.float32),
            pltpu.VMEM((CHUNK, PADEMB), jnp.float32),
            pltpu.VMEM((TABLE_ROWS, PADEMB), jnp.float32),
            pltpu.VMEM_SHARED((TABLE_ROWS, EMB), jnp.float32),
            pltpu.SemaphoreType.DMA,
            pltpu.SemaphoreType.DMA,
            pltpu.SemaphoreType.DMA,
            pltpu.SemaphoreType.DMA,
            pltpu.SemaphoreType.DMA,
            pltpu.SemaphoreType.DMA,
        ],
    )
    def lookup(z_hbm, table_hbm, out_hbm, idx_all, rows0, rows1, vbuf0,
               vbuf1, table_loc, table_sh, gsem0, gsem1, wsem0, wsem1,
               vsem0, vsem1):
        sid = lax.axis_index("s")
        wid = sid * NC + lax.axis_index("c")

        @pl.when(sid == 0)
        def _():
            pltpu.sync_copy(table_hbm, table_sh.at[pl.ds(1, TABLE_ROWS - 1)])

        start = jnp.minimum(wid * SPAN, N_ATOMS - SPAN)
        pltpu.sync_copy(z_hbm.at[pl.ds(start, SPAN)], idx_all)

        plsc.subcore_barrier()

        rows = (rows0, rows1)
        vbuf = (vbuf0, vbuf1)
        gsem = (gsem0, gsem1)
        wsem = (wsem0, wsem1)
        vsem = (vsem0, vsem1)

        def issue_gather(k, b):
            pltpu.async_copy(
                table_sh.at[idx_all.at[pl.ds(k * CHUNK, CHUNK)]],
                rows[b], gsem[b])

        def drain(sem, buf):
            pltpu.make_async_copy(out_hbm.at[pl.ds(0, CHUNK)], buf,
                                  sem).wait()

        def stream_event(e):
            if e >= S_STREAM:
                return
            if e + 1 < S_STREAM:
                if e >= 1:
                    drain(wsem[(e - 1) % 2], rows[(e - 1) % 2])
                issue_gather(e + 1, (e + 1) % 2)
            drain(gsem[e % 2], rows[e % 2])
            pltpu.async_copy(rows[e % 2],
                             out_hbm.at[pl.ds(start + e * CHUNK, CHUNK)],
                             wsem[e % 2])

        def expand_group(kv, g, dst):
            z16 = idx_all[pl.ds(kv * CHUNK + g * GSIZE, GSIZE)]
            rowids = g * GSIZE + lax.iota(jnp.int32, GSIZE)

            @plsc.parallel_loop(0, EMB, 1, unroll=16)
            def _(c):
                cvec = jnp.zeros((GSIZE,), jnp.int32) + c
                vals = plsc.load_gather(table_loc, [z16, cvec])
                plsc.store_scatter(dst, [rowids, cvec], vals)

        def expand_half(kv, h, dst):
            ng = CHUNK // GSIZE

            def body(g, carry):
                expand_group(kv, g, dst)
                return carry

            lax.fori_loop(h * ng // 2, (h + 1) * ng // 2, body, 0)

        issue_gather(0, 0)
        pltpu.sync_copy(table_sh, table_loc.at[:, pl.ds(0, EMB)])

        for i in range(V_CHUNKS):
            kv = S_STREAM + i
            p = i % 2
            if i >= 2:
                drain(vsem[p], vbuf[p].at[:, pl.ds(0, EMB)])
            for e in _events_at_slot(2 * i):
                stream_event(e)
            expand_half(kv, 0, vbuf[p])
            for e in _events_at_slot(2 * i + 1):
                stream_event(e)
            expand_half(kv, 1, vbuf[p])
            pltpu.async_copy(vbuf[p].at[:, pl.ds(0, EMB)],
                             out_hbm.at[pl.ds(start + kv * CHUNK, CHUNK)],
                             vsem[p])


        drain(wsem[(S_STREAM - 2) % 2], rows[(S_STREAM - 2) % 2])
        drain(wsem[(S_STREAM - 1) % 2], rows[(S_STREAM - 1) % 2])
        if V_CHUNKS >= 2:
            drain(vsem[(V_CHUNKS - 2) % 2],
                  vbuf[(V_CHUNKS - 2) % 2].at[:, pl.ds(0, EMB)])
        if V_CHUNKS >= 1:
            drain(vsem[(V_CHUNKS - 1) % 2],
                  vbuf[(V_CHUNKS - 1) % 2].at[:, pl.ds(0, EMB)])

    return lookup


_lookup = _make_lookup()


def kernel(Z, W):
    return _lookup(Z, W)


# speedup vs baseline: 2.0126x; 1.3253x over previous
import functools

import jax
import jax.numpy as jnp
from jax import lax
from jax.experimental import pallas as pl
from jax.experimental.pallas import tpu as pltpu
from jax.experimental.pallas import tpu_sc as plsc

N_ATOMS = 100000
EMB = 128
TABLE_ROWS = 101
CHUNK = 128
GSIZE = 16
PADEMB = EMB + 1

_info = plsc.get_sparse_core_info()
NC = _info.num_cores
NS = _info.num_subcores
NW = NC * NS

CHUNKS_PER_W = -(-N_ATOMS // (CHUNK * NW))
SPAN = CHUNKS_PER_W * CHUNK
S_STREAM = 21
V_CHUNKS = CHUNKS_PER_W - S_STREAM
NGROUP = CHUNK // GSIZE
NSLOTS = V_CHUNKS * NGROUP
NBUF = 3
AHEAD = 2


def _events_at_slot(j):
    return range((S_STREAM * j) // NSLOTS, (S_STREAM * (j + 1)) // NSLOTS)


def _make_lookup():
    mesh = plsc.VectorSubcoreMesh(core_axis_name="c", subcore_axis_name="s")

    @functools.partial(
        pl.kernel,
        mesh=mesh,
        compiler_params=pltpu.CompilerParams(needs_layout_passes=False),
        out_type=jax.ShapeDtypeStruct((N_ATOMS, EMB), jnp.float32),
        scratch_types=[
            pltpu.VMEM((SPAN,), jnp.int32),
            [pltpu.VMEM((CHUNK, EMB), jnp.float32) for _ in range(NBUF)],
            pltpu.VMEM((CHUNK, PADEMB), jnp.float32),
            pltpu.VMEM((TABLE_ROWS, PADEMB), jnp.float32),
            pltpu.VMEM_SHARED((TABLE_ROWS, EMB), jnp.float32),
            [pltpu.SemaphoreType.DMA for _ in range(NBUF)],
            [pltpu.SemaphoreType.DMA for _ in range(NBUF)],
            pltpu.SemaphoreType.DMA,
        ],
    )
    def lookup(z_hbm, table_hbm, out_hbm, idx_all, rows, vbuf, table_loc,
               table_sh, gsem, wsem, vsem):
        sid = lax.axis_index("s")
        wid = sid * NC + lax.axis_index("c")

        @pl.when(sid == 0)
        def _():
            pltpu.sync_copy(table_hbm, table_sh.at[pl.ds(1, TABLE_ROWS - 1)])

        start = jnp.minimum(wid * SPAN, N_ATOMS - SPAN)
        pltpu.sync_copy(z_hbm.at[pl.ds(start, SPAN)], idx_all)

        plsc.subcore_barrier()

        def issue_gather(k, b):
            pltpu.async_copy(
                table_sh.at[idx_all.at[pl.ds(k * CHUNK, CHUNK)]],
                rows[b], gsem[b])

        def drain(sem, buf):
            pltpu.make_async_copy(out_hbm.at[pl.ds(0, CHUNK)], buf,
                                  sem).wait()

        def stream_event(e):
            if e >= S_STREAM:
                return
            if e + AHEAD < S_STREAM:
                b1 = (e + AHEAD) % NBUF
                if e + AHEAD >= NBUF:
                    drain(wsem[b1], rows[b1])
                issue_gather(e + AHEAD, b1)
            b = e % NBUF
            drain(gsem[b], rows[b])
            pltpu.async_copy(rows[b],
                             out_hbm.at[pl.ds(start + e * CHUNK, CHUNK)],
                             wsem[b])

        def expand_group(kv, g, dst):
            z16 = idx_all[pl.ds(kv * CHUNK + g * GSIZE, GSIZE)]
            rowids = g * GSIZE + lax.iota(jnp.int32, GSIZE)

            @plsc.parallel_loop(0, EMB, 1, unroll=8)
            def _(c):
                cvec = jnp.zeros((GSIZE,), jnp.int32) + c
                vals = plsc.load_gather(table_loc, [z16, cvec])
                plsc.store_scatter(dst, [rowids, cvec], vals)

        for k in range(AHEAD):
            issue_gather(k, k % NBUF)
        pltpu.sync_copy(table_sh, table_loc.at[:, pl.ds(0, EMB)])

        for i in range(V_CHUNKS):
            kv = S_STREAM + i
            if i >= 1:
                drain(vsem, vbuf.at[:, pl.ds(0, EMB)])
            for g in range(NGROUP):
                for e in _events_at_slot(i * NGROUP + g):
                    stream_event(e)
                expand_group(kv, g, vbuf)
            pltpu.async_copy(vbuf.at[:, pl.ds(0, EMB)],
                             out_hbm.at[pl.ds(start + kv * CHUNK, CHUNK)],
                             vsem)

        for e in range(max(0, S_STREAM - NBUF), S_STREAM):
            drain(wsem[e % NBUF], rows[e % NBUF])
        if V_CHUNKS >= 1:
            drain(vsem, vbuf.at[:, pl.ds(0, EMB)])

    return lookup


_lookup = _make_lookup()


def kernel(Z, W):
    return _lookup(Z, W)


# speedup vs baseline: 2.7779x; 1.3802x over previous
import functools

import jax
import jax.numpy as jnp
from jax import lax
from jax.experimental import pallas as pl
from jax.experimental.pallas import tpu as pltpu
from jax.experimental.pallas import tpu_sc as plsc

N_ATOMS = 100000
EMB = 128
TABLE_ROWS = 101
CHUNK = 128

_info = plsc.get_sparse_core_info()
NC = _info.num_cores
NS = _info.num_subcores
NW = NC * NS

CHUNKS_PER_W = -(-N_ATOMS // (CHUNK * NW))
SPAN = CHUNKS_PER_W * CHUNK
NBUF = 3
AHEAD = 2


def _make_lookup():
    mesh = plsc.VectorSubcoreMesh(core_axis_name="c", subcore_axis_name="s")

    @functools.partial(
        pl.kernel,
        mesh=mesh,
        compiler_params=pltpu.CompilerParams(needs_layout_passes=False),
        out_type=jax.ShapeDtypeStruct((N_ATOMS, EMB), jnp.float32),
        scratch_types=[
            pltpu.VMEM((SPAN,), jnp.int32),
            [pltpu.VMEM((CHUNK, EMB), jnp.float32) for _ in range(NBUF)],
            pltpu.VMEM_SHARED((TABLE_ROWS, EMB), jnp.float32),
            [pltpu.SemaphoreType.DMA for _ in range(NBUF)],
            [pltpu.SemaphoreType.DMA for _ in range(NBUF)],
        ],
    )
    def lookup(z_hbm, table_hbm, out_hbm, idx_all, rows, table_sh, gsem,
               wsem):
        sid = lax.axis_index("s")
        wid = sid * NC + lax.axis_index("c")

        @pl.when(sid == 0)
        def _():
            pltpu.sync_copy(table_hbm, table_sh.at[pl.ds(1, TABLE_ROWS - 1)])

        start = jnp.minimum(wid * SPAN, N_ATOMS - SPAN)
        pltpu.sync_copy(z_hbm.at[pl.ds(start, SPAN)], idx_all)

        plsc.subcore_barrier()

        def issue_gather(k, b):
            pltpu.async_copy(
                table_sh.at[idx_all.at[pl.ds(k * CHUNK, CHUNK)]],
                rows[b], gsem[b])

        def drain(sem, buf):
            pltpu.make_async_copy(out_hbm.at[pl.ds(0, CHUNK)], buf,
                                  sem).wait()

        for k in range(AHEAD):
            issue_gather(k, k % NBUF)

        for e in range(CHUNKS_PER_W):
            if e + AHEAD < CHUNKS_PER_W:
                b1 = (e + AHEAD) % NBUF
                if e + AHEAD >= NBUF:
                    drain(wsem[b1], rows[b1])
                issue_gather(e + AHEAD, b1)
            b = e % NBUF
            drain(gsem[b], rows[b])
            pltpu.async_copy(rows[b],
                             out_hbm.at[pl.ds(start + e * CHUNK, CHUNK)],
                             wsem[b])

        for e in range(CHUNKS_PER_W - NBUF, CHUNKS_PER_W):
            drain(wsem[e % NBUF], rows[e % NBUF])

    return lookup


_lookup = _make_lookup()


def kernel(Z, W):
    return _lookup(Z, W)
